# baseline (speedup 1.0000x reference)
import functools

import jax
import jax.numpy as jnp
from jax import lax
from jax.experimental import pallas as pl
from jax.experimental.pallas import tpu as pltpu
from jax.experimental.pallas import tpu_sc as plsc

D = H = W = 128
C = 16
NPTS = 64 * 64 * 64
LO = 63
SUB = 65
SROWS = SUB * SUB * SUB

NC, NS, L = 2, 16, 16
NW = NC * NS
PPT = NPTS // NW
CHUNK = 256
NG = CHUNK // L
NCHUNKS = PPT // CHUNK

_SPLAT_DNUMS = lax.GatherDimensionNumbers(
    offset_dims=(), collapsed_slice_dims=(0,), start_index_map=(0,))


def _lane_splat(vec, lane):
  idx = jnp.full((L, 1), lane, jnp.int32)
  return lax.gather(vec, idx, _SPLAT_DNUMS, (1,),
                    mode=lax.GatherScatterMode.PROMISE_IN_BOUNDS)


def _lane_perm(vec, idx):
  return lax.gather(vec, idx[:, None], _SPLAT_DNUMS, (1,),
                    mode=lax.GatherScatterMode.PROMISE_IN_BOUNDS)


def _transpose16(vs, lanes):
  for b in (8, 4, 2, 1):
    perm = jnp.bitwise_xor(lanes, b)
    mask = jnp.bitwise_and(lanes, b) != 0
    new = [None] * L
    for i in range(L):
      g = _lane_perm(vs[i ^ b], perm)
      if i & b == 0:
        new[i] = jnp.where(mask, g, vs[i])
      else:
        new[i] = jnp.where(mask, vs[i], g)
    vs = new
  return vs


NZY = SUB * SUB
NPAIR = 67
_XS = (48, 64, 80, 96, 112)


def _sc_linearize(param_t):
  mesh = plsc.VectorSubcoreMesh(core_axis_name="c", subcore_axis_name="s")

  @functools.partial(
      pl.kernel,
      mesh=mesh,
      compiler_params=pltpu.CompilerParams(use_tc_tiling_on_sc=False),
      out_type=jax.ShapeDtypeStruct((SROWS * C,), jnp.float32),
      scratch_types=[
          pltpu.VMEM((C, 128), jnp.float32),
          pltpu.VMEM((C, 128), jnp.float32),
          pltpu.VMEM((SUB * C,), jnp.float32),
          pltpu.VMEM((SUB * C,), jnp.float32),
          pltpu.SemaphoreType.DMA,
          pltpu.SemaphoreType.DMA,
          pltpu.SemaphoreType.DMA,
          pltpu.SemaphoreType.DMA,
      ],
  )
  def k(p_hbm, t_hbm, lb0, lb1, ob0, ob1, is0, is1, os0, os1):
    wid = lax.axis_index("s") * NC + lax.axis_index("c")
    iota = lax.iota(jnp.int32, L)

    def fire_in(j, lb, isem):
      zy = wid + NW * j

      @pl.when(zy < NZY)
      def _():
        z = zy // SUB
        y = zy % SUB
        pltpu.async_copy(p_hbm.at[z + LO, y + LO], lb, isem)

    def step(j, lb, ob, isem, osem):
      zy = wid + NW * j

      @pl.when(zy < NZY)
      def _():
        pltpu.make_async_copy(p_hbm.at[LO, LO], lb, isem).wait()

      @pl.when((j >= 2) & (zy - 2 * NW < NZY))
      def _():
        pltpu.make_async_copy(t_hbm.at[pl.ds(0, SUB * C)], ob, osem).wait()

      @pl.when(zy < NZY)
      def _():
        for xs in _XS:
          vs = [lb[c, pl.ds(xs, L)] for c in range(C)]
          vs = _transpose16(vs, iota)
          for l in range(L):
            x = xs + l
            if LO <= x <= 127:
              ob[pl.ds((x - LO) * C, C)] = vs[l]
        pltpu.async_copy(ob, t_hbm.at[pl.ds(zy * SUB * C, SUB * C)], osem)

      fire_in(j + 2, lb, isem)

    fire_in(0, lb0, is0)
    fire_in(1, lb1, is1)

    def pair_body(i, carry):
      step(2 * i, lb0, ob0, is0, os0)
      step(2 * i + 1, lb1, ob1, is1, os1)
      return carry

    lax.fori_loop(0, NPAIR, pair_body, 0)

    @pl.when(wid + NW * 132 < NZY)
    def _():
      pltpu.make_async_copy(t_hbm.at[pl.ds(0, SUB * C)], ob0, os0).wait()

    @pl.when(wid + NW * 133 < NZY)
    def _():
      pltpu.make_async_copy(t_hbm.at[pl.ds(0, SUB * C)], ob1, os1).wait()

  return k(param_t)


def _sc_interp(grid4, table):
  mesh = plsc.VectorSubcoreMesh(core_axis_name="c", subcore_axis_name="s")

  @functools.partial(
      pl.kernel,
      mesh=mesh,
      compiler_params=pltpu.CompilerParams(use_tc_tiling_on_sc=False),
      out_type=jax.ShapeDtypeStruct((64 * 64 * C * 64,), jnp.float32),
      scratch_types=[
          pltpu.VMEM((2, 3, 64, 64), jnp.float32),
          pltpu.VMEM((NG, 128), jnp.int32),
          pltpu.VMEM((NG, 128), jnp.int32),
          pltpu.VMEM((NG * 48,), jnp.float32),
          pltpu.VMEM((NG * 48,), jnp.float32),
          pltpu.VMEM((NG * 128, C), jnp.float32),
          pltpu.VMEM((NG * 128, C), jnp.float32),
          pltpu.VMEM((4 * C * 64,), jnp.float32),
          pltpu.SemaphoreType.DMA,
          pltpu.SemaphoreType.DMA,
      ],
  )
  def k(grid_hbm, tab_hbm, out_hbm,
        gv, idx0, idx1, w0, w1, rows0, rows1, outv, sem0, sem1):
    wid = lax.axis_index("s") * NC + lax.axis_index("c")
    base = wid * PPT
    iota = lax.iota(jnp.int32, L)
    pltpu.sync_copy(grid_hbm.at[pl.ds(wid * 2, 2)], gv)


    def gen_fire(ci, idxv, wv, rowsv, sem):
      c0 = ci * CHUNK

      def gen_body(g, carry):
        lp = c0 + g * L
        plane = lp // 4096
        row = (lp // 64) % 64
        xs = lp % 64
        z = gv[plane, 0, row, pl.ds(xs, L)]
        y = gv[plane, 1, row, pl.ds(xs, L)]
        x = gv[plane, 2, row, pl.ds(xs, L)]
        fz = (z + 1.0) * 0.5 * (D - 1)
        fy = (y + 1.0) * 0.5 * (H - 1)
        fx = (x + 1.0) * 0.5 * (W - 1)
        z0 = fz.astype(jnp.int32)
        y0 = fy.astype(jnp.int32)
        x0 = fx.astype(jnp.int32)
        wz = fz - z0.astype(jnp.float32)
        wy = fy - y0.astype(jnp.float32)
        wx = fx - x0.astype(jnp.float32)
        z0 = z0 - LO
        y0 = y0 - LO
        x0 = x0 - LO
        z1 = jnp.minimum(z0 + 1, SUB - 1)
        y1 = jnp.minimum(y0 + 1, SUB - 1)
        x1 = jnp.minimum(x0 + 1, SUB - 1)
        za = z0 * (SUB * SUB)
        zb = z1 * (SUB * SUB)
        ya = y0 * SUB
        yb = y1 * SUB
        kk = 0
        for zy in (za + ya, za + yb, zb + ya, zb + yb):
          for xi in (x0, x1):
            idxv[g, pl.ds(kk * L, L)] = zy + xi
            kk += 1
        wv[pl.ds(g * 48, L)] = wx
        wv[pl.ds(g * 48 + L, L)] = wy
        wv[pl.ds(g * 48 + 2 * L, L)] = wz
        pltpu.async_copy(tab_hbm.at[idxv.at[g]],
                         rowsv.at[pl.ds(g * 128, 128)], sem)
        return carry

      lax.fori_loop(0, NG, gen_body, 0)

    def drain(rowsv, sem):
      pltpu.make_async_copy(tab_hbm.at[pl.ds(0, NG * 128)], rowsv, sem).wait()

    def fma_out(ci, wv, rowsv):
      def fma_body(g, carry):
        goff = g * 128
        obase = (g // 4) * (C * 64) + (g % 4) * L
        wxv = wv[pl.ds(g * 48, L)]
        wyv = wv[pl.ds(g * 48 + L, L)]
        wzv = wv[pl.ds(g * 48 + 2 * L, L)]
        accs = []
        for p in range(L):
          sx = _lane_splat(wxv, p)
          sy = _lane_splat(wyv, p)
          sz = _lane_splat(wzv, p)
          r = [rowsv[goff + kk * L + p, :] for kk in range(8)]
          tx = [r[2 * j] + sx * (r[2 * j + 1] - r[2 * j]) for j in range(4)]
          ty = [tx[2 * j] + sy * (tx[2 * j + 1] - tx[2 * j]) for j in range(2)]
          accs.append(ty[0] + sz * (ty[1] - ty[0]))
        accs = _transpose16(accs, iota)
        for c in range(C):
          outv[pl.ds(obase + c * 64, L)] = accs[c]
        return carry

      lax.fori_loop(0, NG, fma_body, 0)
      off = ((base + ci * CHUNK) // 64) * (C * 64)
      pltpu.sync_copy(outv, out_hbm.at[pl.ds(off, 4 * C * 64)])

    gen_fire(0, idx0, w0, rows0, sem0)
    gen_fire(1, idx1, w1, rows1, sem1)

    def pair_body(i, carry):
      even = 2 * i
      drain(rows0, sem0)
      fma_out(even, w0, rows0)

      @pl.when(i < NCHUNKS // 2 - 1)
      def _():
        gen_fire(even + 2, idx0, w0, rows0, sem0)

      drain(rows1, sem1)
      fma_out(even + 1, w1, rows1)

      @pl.when(i < NCHUNKS // 2 - 1)
      def _():
        gen_fire(even + 3, idx1, w1, rows1, sem1)

      return carry

    lax.fori_loop(0, NCHUNKS // 2, pair_body, 0)

  return k(grid4, table)


def kernel(grid, param):
  grid4 = jnp.transpose(grid, (0, 3, 1, 2))
  param_t = jnp.transpose(param, (0, 1, 3, 2))
  table = _sc_linearize(param_t).reshape(SROWS, C)
  out1 = _sc_interp(grid4, table)
  return jnp.transpose(out1.reshape(64, 64, C, 64), (0, 1, 3, 2))


# speedup vs baseline: 1.0317x; 1.0317x over previous
import functools

import jax
import jax.numpy as jnp
from jax import lax
from jax.experimental import pallas as pl
from jax.experimental.pallas import tpu as pltpu
from jax.experimental.pallas import tpu_sc as plsc

D = H = W = 128
C = 16
NPTS = 64 * 64 * 64
LO = 63
SUB = 65
SROWS = SUB * SUB * SUB

NC, NS, L = 2, 16, 16
NW = NC * NS
PPT = NPTS // NW
CHUNK = 256
NG = CHUNK // L
NCHUNKS = PPT // CHUNK

_SPLAT_DNUMS = lax.GatherDimensionNumbers(
    offset_dims=(), collapsed_slice_dims=(0,), start_index_map=(0,))


def _lane_splat(vec, lane):
  idx = jnp.full((L, 1), lane, jnp.int32)
  return lax.gather(vec, idx, _SPLAT_DNUMS, (1,),
                    mode=lax.GatherScatterMode.PROMISE_IN_BOUNDS)


def _lane_perm(vec, idx):
  return lax.gather(vec, idx[:, None], _SPLAT_DNUMS, (1,),
                    mode=lax.GatherScatterMode.PROMISE_IN_BOUNDS)


def _transpose16(vs, lanes):
  for b in (8, 4, 2, 1):
    perm = jnp.bitwise_xor(lanes, b)
    mask = jnp.bitwise_and(lanes, b) != 0
    new = [None] * L
    for i in range(L):
      g = _lane_perm(vs[i ^ b], perm)
      if i & b == 0:
        new[i] = jnp.where(mask, g, vs[i])
      else:
        new[i] = jnp.where(mask, vs[i], g)
    vs = new
  return vs


NZY = SUB * SUB
NPAIR = 67
_XS = (48, 64, 80, 96, 112)


def _sc_linearize(param_t):
  mesh = plsc.VectorSubcoreMesh(core_axis_name="c", subcore_axis_name="s")

  @functools.partial(
      pl.kernel,
      mesh=mesh,
      compiler_params=pltpu.CompilerParams(use_tc_tiling_on_sc=False),
      out_type=jax.ShapeDtypeStruct((SROWS * 2 * C,), jnp.float32),
      scratch_types=[
          pltpu.VMEM((C, 128), jnp.float32),
          pltpu.VMEM((C, 128), jnp.float32),
          pltpu.VMEM((SUB * 2 * C,), jnp.float32),
          pltpu.VMEM((SUB * 2 * C,), jnp.float32),
          pltpu.SemaphoreType.DMA,
          pltpu.SemaphoreType.DMA,
          pltpu.SemaphoreType.DMA,
          pltpu.SemaphoreType.DMA,
      ],
  )
  def k(p_hbm, t_hbm, lb0, lb1, ob0, ob1, is0, is1, os0, os1):
    wid = lax.axis_index("s") * NC + lax.axis_index("c")
    iota = lax.iota(jnp.int32, L)

    def fire_in(j, lb, isem):
      zy = wid + NW * j

      @pl.when(zy < NZY)
      def _():
        z = zy // SUB
        y = zy % SUB
        pltpu.async_copy(p_hbm.at[z + LO, y + LO], lb, isem)

    def step(j, lb, ob, isem, osem):
      zy = wid + NW * j

      @pl.when(zy < NZY)
      def _():
        pltpu.make_async_copy(p_hbm.at[LO, LO], lb, isem).wait()

      @pl.when((j >= 2) & (zy - 2 * NW < NZY))
      def _():
        pltpu.make_async_copy(t_hbm.at[pl.ds(0, SUB * 2 * C)], ob, osem).wait()

      @pl.when(zy < NZY)
      def _():
        for xs in _XS:
          vs = [lb[c, pl.ds(xs, L)] for c in range(C)]
          vs = _transpose16(vs, iota)
          for l in range(L):
            x = xs + l
            if LO <= x <= 127:
              ob[pl.ds((x - LO) * 2 * C, C)] = vs[l]
              if x > LO:
                ob[pl.ds((x - LO - 1) * 2 * C + C, C)] = vs[l]
              if x == 127:
                ob[pl.ds((x - LO) * 2 * C + C, C)] = vs[l]
        pltpu.async_copy(ob, t_hbm.at[pl.ds(zy * SUB * 2 * C, SUB * 2 * C)],
                         osem)

      fire_in(j + 2, lb, isem)

    fire_in(0, lb0, is0)
    fire_in(1, lb1, is1)

    def pair_body(i, carry):
      step(2 * i, lb0, ob0, is0, os0)
      step(2 * i + 1, lb1, ob1, is1, os1)
      return carry

    lax.fori_loop(0, NPAIR, pair_body, 0)

    @pl.when(wid + NW * 132 < NZY)
    def _():
      pltpu.make_async_copy(t_hbm.at[pl.ds(0, SUB * 2 * C)], ob0, os0).wait()

    @pl.when(wid + NW * 133 < NZY)
    def _():
      pltpu.make_async_copy(t_hbm.at[pl.ds(0, SUB * 2 * C)], ob1, os1).wait()

  return k(param_t)


def _sc_interp(grid4, table):
  mesh = plsc.VectorSubcoreMesh(core_axis_name="c", subcore_axis_name="s")

  @functools.partial(
      pl.kernel,
      mesh=mesh,
      compiler_params=pltpu.CompilerParams(use_tc_tiling_on_sc=False),
      out_type=jax.ShapeDtypeStruct((64 * 64 * C * 64,), jnp.float32),
      scratch_types=[
          pltpu.VMEM((2, 3, 64, 64), jnp.float32),
          pltpu.VMEM((NG, 64), jnp.int32),
          pltpu.VMEM((NG, 64), jnp.int32),
          pltpu.VMEM((NG * 48,), jnp.float32),
          pltpu.VMEM((NG * 48,), jnp.float32),
          pltpu.VMEM((NG * 64, 2 * C), jnp.float32),
          pltpu.VMEM((NG * 64, 2 * C), jnp.float32),
          pltpu.VMEM((4 * C * 64,), jnp.float32),
          pltpu.SemaphoreType.DMA,
          pltpu.SemaphoreType.DMA,
      ],
  )
  def k(grid_hbm, tab_hbm, out_hbm,
        gv, idx0, idx1, w0, w1, rows0, rows1, outv, sem0, sem1):
    wid = lax.axis_index("s") * NC + lax.axis_index("c")
    base = wid * PPT
    iota = lax.iota(jnp.int32, L)
    pltpu.sync_copy(grid_hbm.at[pl.ds(wid * 2, 2)], gv)


    def gen_fire(ci, idxv, wv, rowsv, sem):
      c0 = ci * CHUNK

      def gen_body(g, carry):
        lp = c0 + g * L
        plane = lp // 4096
        row = (lp // 64) % 64
        xs = lp % 64
        z = gv[plane, 0, row, pl.ds(xs, L)]
        y = gv[plane, 1, row, pl.ds(xs, L)]
        x = gv[plane, 2, row, pl.ds(xs, L)]
        fz = (z + 1.0) * 0.5 * (D - 1)
        fy = (y + 1.0) * 0.5 * (H - 1)
        fx = (x + 1.0) * 0.5 * (W - 1)
        z0 = fz.astype(jnp.int32)
        y0 = fy.astype(jnp.int32)
        x0 = fx.astype(jnp.int32)
        wz = fz - z0.astype(jnp.float32)
        wy = fy - y0.astype(jnp.float32)
        wx = fx - x0.astype(jnp.float32)
        z0 = z0 - LO
        y0 = y0 - LO
        x0 = x0 - LO
        z1 = jnp.minimum(z0 + 1, SUB - 1)
        y1 = jnp.minimum(y0 + 1, SUB - 1)
        za = z0 * (SUB * SUB)
        zb = z1 * (SUB * SUB)
        ya = y0 * SUB
        yb = y1 * SUB
        kk = 0
        for zy in (za + ya, za + yb, zb + ya, zb + yb):
          idxv[g, pl.ds(kk * L, L)] = zy + x0
          kk += 1
        wv[pl.ds(g * 48, L)] = wx
        wv[pl.ds(g * 48 + L, L)] = wy
        wv[pl.ds(g * 48 + 2 * L, L)] = wz
        pltpu.async_copy(tab_hbm.at[idxv.at[g]],
                         rowsv.at[pl.ds(g * 64, 64)], sem)
        return carry

      lax.fori_loop(0, NG, gen_body, 0)

    def drain(rowsv, sem):
      pltpu.make_async_copy(tab_hbm.at[pl.ds(0, NG * 64)], rowsv, sem).wait()

    def fma_out(ci, wv, rowsv):
      def fma_body(g, carry):
        goff = g * 64
        obase = (g // 4) * (C * 64) + (g % 4) * L
        wxv = wv[pl.ds(g * 48, L)]
        wyv = wv[pl.ds(g * 48 + L, L)]
        wzv = wv[pl.ds(g * 48 + 2 * L, L)]
        accs = []
        for p in range(L):
          sx = _lane_splat(wxv, p)
          sy = _lane_splat(wyv, p)
          sz = _lane_splat(wzv, p)
          tx = []
          for j in range(4):
            o = goff + j * L + p
            r0 = rowsv[o, pl.ds(0, C)]
            r1 = rowsv[o, pl.ds(C, C)]
            tx.append(r0 + sx * (r1 - r0))
          ty = [tx[2 * j] + sy * (tx[2 * j + 1] - tx[2 * j]) for j in range(2)]
          accs.append(ty[0] + sz * (ty[1] - ty[0]))
        accs = _transpose16(accs, iota)
        for c in range(C):
          outv[pl.ds(obase + c * 64, L)] = accs[c]
        return carry

      lax.fori_loop(0, NG, fma_body, 0)
      off = ((base + ci * CHUNK) // 64) * (C * 64)
      pltpu.sync_copy(outv, out_hbm.at[pl.ds(off, 4 * C * 64)])

    gen_fire(0, idx0, w0, rows0, sem0)
    gen_fire(1, idx1, w1, rows1, sem1)

    def pair_body(i, carry):
      even = 2 * i
      drain(rows0, sem0)
      fma_out(even, w0, rows0)

      @pl.when(i < NCHUNKS // 2 - 1)
      def _():
        gen_fire(even + 2, idx0, w0, rows0, sem0)

      drain(rows1, sem1)
      fma_out(even + 1, w1, rows1)

      @pl.when(i < NCHUNKS // 2 - 1)
      def _():
        gen_fire(even + 3, idx1, w1, rows1, sem1)

      return carry

    lax.fori_loop(0, NCHUNKS // 2, pair_body, 0)

  return k(grid4, table)


def kernel(grid, param):
  grid4 = jnp.transpose(grid, (0, 3, 1, 2))
  param_t = jnp.transpose(param, (0, 1, 3, 2))
  table = _sc_linearize(param_t).reshape(SROWS, 2 * C)
  out1 = _sc_interp(grid4, table)
  return jnp.transpose(out1.reshape(64, 64, C, 64), (0, 1, 3, 2))
